# baseline (speedup 1.0000x reference)
import functools

import jax
import jax.numpy as jnp
from jax import lax
from jax.experimental import pallas as pl
from jax.experimental.pallas import tpu as pltpu
from jax.experimental.pallas import tpu_sc as plsc

D = 128
CH = 128


def _make_sc_gather(n_rows_table, b_pad):
    info = plsc.get_sparse_core_info()
    nc, ns = info.num_cores, info.num_subcores
    nw = nc * ns
    assert b_pad % (nw * CH * 2) == 0
    rows_per_w = b_pad // nw
    cpw = rows_per_w // CH

    mesh = plsc.VectorSubcoreMesh(core_axis_name="c", subcore_axis_name="s")

    @functools.partial(
        pl.kernel,
        mesh=mesh,
        out_type=jax.ShapeDtypeStruct((b_pad, D), jnp.float32),
        scratch_types=[
            pltpu.VMEM((CH,), jnp.int32),
            pltpu.VMEM((CH,), jnp.int32),
            pltpu.VMEM((CH, D), jnp.float32),
            pltpu.VMEM((CH, D), jnp.float32),
            pltpu.SemaphoreType.DMA,
            pltpu.SemaphoreType.DMA,
        ],
    )
    def gather_k(table, idxh, out, ia, ib, ra, rb, sa, sb):
        wid = lax.axis_index("s") * nc + lax.axis_index("c")
        base = wid * rows_per_w

        def load_idx(iref, j):
            pltpu.sync_copy(idxh.at[pl.ds(base + j * CH, CH)], iref)

        def start_gather(iref, rref, sem):
            pltpu.make_async_copy(table.at[iref], rref, sem).start()

        def wait_gather(iref, rref, sem):
            pltpu.make_async_copy(table.at[iref], rref, sem).wait()

        def store(rref, j):
            pltpu.sync_copy(rref, out.at[pl.ds(base + j * CH, CH)])

        load_idx(ia, 0)
        start_gather(ia, ra, sa)

        def pair(p, carry):
            j0 = p * 2
            j1 = j0 + 1
            load_idx(ib, j1)
            start_gather(ib, rb, sb)
            wait_gather(ia, ra, sa)
            store(ra, j0)
            @pl.when(j0 + 2 < cpw)
            def _():
                load_idx(ia, j0 + 2)
                start_gather(ia, ra, sa)
            wait_gather(ib, rb, sb)
            store(rb, j1)
            return carry

        lax.fori_loop(0, cpw // 2, pair, 0)

    return gather_k


def _mlp_body(x_ref, w1_ref, b1_ref, w2_ref, b2_ref, o_ref):
    x = x_ref[...]
    h = jnp.dot(x, w1_ref[...], preferred_element_type=jnp.float32) + b1_ref[...]
    m = h * jnp.tanh(jax.nn.softplus(h))
    y = x + jnp.dot(m, w2_ref[...], preferred_element_type=jnp.float32) + b2_ref[...]
    o_ref[...] = y


def _mlp_call(g_view, w1, b1, w2, b2, n_tup, off_blocks, bt):
    s = w1.shape[0]
    return pl.pallas_call(
        _mlp_body,
        grid=(n_tup // bt,),
        in_specs=[
            pl.BlockSpec((bt, s), lambda i: (i + off_blocks, 0)),
            pl.BlockSpec((s, s), lambda i: (0, 0)),
            pl.BlockSpec((1, s), lambda i: (0, 0)),
            pl.BlockSpec((s, s), lambda i: (0, 0)),
            pl.BlockSpec((1, s), lambda i: (0, 0)),
        ],
        out_specs=pl.BlockSpec((bt, s), lambda i: (i, 0)),
        out_shape=jax.ShapeDtypeStruct((n_tup, s), jnp.float32),
    )(g_view, w1, b1.reshape(1, s), w2, b2.reshape(1, s))


def kernel(node_embeddings, rel_a, rel_b, rel_c, rel_d,
           W1_rel_a, b1_rel_a, W2_rel_a, b2_rel_a,
           W1_rel_b, b1_rel_b, W2_rel_b, b2_rel_b,
           W1_rel_c, b1_rel_c, W2_rel_c, b2_rel_c,
           W1_rel_d, b1_rel_d, W2_rel_d, b2_rel_d):
    idxs = [rel_a, rel_b, rel_c, rel_d]
    pars = [
        (W1_rel_a, b1_rel_a, W2_rel_a, b2_rel_a),
        (W1_rel_b, b1_rel_b, W2_rel_b, b2_rel_b),
        (W1_rel_c, b1_rel_c, W2_rel_c, b2_rel_c),
        (W1_rel_d, b1_rel_d, W2_rel_d, b2_rel_d),
    ]
    total = sum(int(ix.shape[0]) for ix in idxs)

    info = plsc.get_sparse_core_info()
    nw = info.num_cores * info.num_subcores
    unit = nw * CH * 2
    b_pad = ((total + unit - 1) // unit) * unit
    while b_pad % 3 != 0:
        b_pad += unit

    idx_all = jnp.concatenate(idxs)
    idx_all = jnp.pad(idx_all, (0, b_pad - total))

    g = _make_sc_gather(node_embeddings.shape[0], b_pad)(node_embeddings, idx_all)

    bt = 1000
    msgs = []
    off_rows = 0
    for ix, (w1, b1, w2, b2) in zip(idxs, pars):
        s = int(w1.shape[0])
        a = s // D
        n_tup = int(ix.shape[0]) // a
        view = g.reshape(b_pad * D // s, s)
        off_v = off_rows // a
        assert off_v % bt == 0 and n_tup % bt == 0
        y = _mlp_call(view, w1, b1, w2, b2, n_tup, off_v // bt, bt)
        msgs.append(y.reshape(-1, D))
        off_rows += n_tup * a

    output_messages = jnp.concatenate(msgs, axis=0)
    output_indices = jnp.concatenate(idxs)
    return output_messages, output_indices
